# Initial kernel scaffold; baseline (speedup 1.0000x reference)
import jax
import jax.numpy as jnp
from jax.experimental import pallas as pl


def kernel(x, t, edge_index, node_embedding, te_W1, te_b1, te_W2, te_b2, film_W, film_b, ln_g, ln_b, e_W1, e_b1, e_W2, e_b2, n_W1, n_b1, n_W2, n_b2, ch_W1, ch_b1, ch_W2):
    raise NotImplementedError("write your pallas kernel here")


# speedup vs baseline: 2.9708x; 2.9708x over previous
import functools

import jax
import jax.numpy as jnp
from jax import lax
from jax.experimental import pallas as pl
from jax.experimental.pallas import tpu as pltpu
from jax.experimental.pallas import tpu_sc as plsc

_N = 10000
_E = 320000
_DIM = 128
_DEPTH = 6
_K = 32.0
_WT = 10.0

_NC = 2
_NS = 16
_NW = _NC * _NS
_EW = _E // _NW

_CG = 80
_NCG = _EW // _CG
_CS = 40
_NCS = _EW // _CS
_CSV = 80
_NCSV = _EW // _CSV
_NP = 10240
_NPT = _NP // _NS

_BE = 2560
_BN = 2000

_f32 = jnp.float32


def _sc_mesh():
    return plsc.VectorSubcoreMesh(core_axis_name="c", subcore_axis_name="s")


_SC_PARAMS = pltpu.CompilerParams(needs_layout_passes=False)


def _wid():
    return lax.axis_index("s") * _NC + lax.axis_index("c")


def _geom_body(x0, x1, row, col, rij0, rij1, dsq,
               x0_v, x1_v, row_v, col_v, o0_v, o1_v, od_v):
    base = pl.multiple_of(_wid() * _EW, 8)
    pltpu.sync_copy(x0, x0_v)
    pltpu.sync_copy(x1, x1_v)
    pltpu.sync_copy(row.at[pl.ds(base, _EW)], row_v)
    pltpu.sync_copy(col.at[pl.ds(base, _EW)], col_v)

    def step(i, carry):
        o = pl.multiple_of(i * 16, 8)
        ir = row_v[pl.ds(o, 16)]
        ic = col_v[pl.ds(o, 16)]
        r0 = plsc.load_gather(x0_v, [ir]) - plsc.load_gather(x0_v, [ic])
        r1 = plsc.load_gather(x1_v, [ir]) - plsc.load_gather(x1_v, [ic])
        o0_v[pl.ds(o, 16)] = r0
        o1_v[pl.ds(o, 16)] = r1
        od_v[pl.ds(o, 16)] = r0 * r0 + r1 * r1
        return carry

    lax.fori_loop(0, _EW // 16, step, 0)
    pltpu.sync_copy(o0_v, rij0.at[pl.ds(base, _EW)])
    pltpu.sync_copy(o1_v, rij1.at[pl.ds(base, _EW)])
    pltpu.sync_copy(od_v, dsq.at[pl.ds(base, _EW)])


def _geom(x0, x1, row, col):
    f = pl.kernel(
        _geom_body,
        out_type=(jax.ShapeDtypeStruct((_E,), _f32),) * 3,
        mesh=_sc_mesh(),
        compiler_params=_SC_PARAMS,
        scratch_types=[
            pltpu.VMEM((_N,), _f32), pltpu.VMEM((_N,), _f32),
            pltpu.VMEM((_EW,), jnp.int32), pltpu.VMEM((_EW,), jnp.int32),
            pltpu.VMEM((_EW,), _f32), pltpu.VMEM((_EW,), _f32),
            pltpu.VMEM((_EW,), _f32),
        ],
    )
    return f(x0, x1, row, col)


def _gather_body(t1, t2, row, col, g1, g2, row_v, col_v, b1, b2, s1, s2):
    base = pl.multiple_of(_wid() * _EW, 8)
    pltpu.sync_copy(row.at[pl.ds(base, _EW)], row_v)
    pltpu.sync_copy(col.at[pl.ds(base, _EW)], col_v)

    def chunk(g, carry):
        o = pl.multiple_of(g * _CG, 8)
        d1 = pltpu.async_copy(t1.at[row_v.at[pl.ds(o, _CG)]], b1, s1)
        d2 = pltpu.async_copy(t2.at[col_v.at[pl.ds(o, _CG)]], b2, s2)
        d1.wait()
        d2.wait()
        pltpu.sync_copy(b1, g1.at[pl.ds(base + o, _CG)])
        pltpu.sync_copy(b2, g2.at[pl.ds(base + o, _CG)])
        return carry

    lax.fori_loop(0, _NCG, chunk, 0)


def _gather2(t1, t2, row, col):
    f = pl.kernel(
        _gather_body,
        out_type=(jax.ShapeDtypeStruct((_E, _DIM), _f32),) * 2,
        mesh=_sc_mesh(),
        compiler_params=_SC_PARAMS,
        scratch_types=[
            pltpu.VMEM((_EW,), jnp.int32), pltpu.VMEM((_EW,), jnp.int32),
            pltpu.VMEM((_CG, _DIM), _f32), pltpu.VMEM((_CG, _DIM), _f32),
            pltpu.SemaphoreType.DMA, pltpu.SemaphoreType.DMA,
        ],
    )
    return f(t1, t2, row, col)


def _make_scatter_body(width, cs, ncs):
    def body(m, idx3, zeros, parts, idx_v, mb, acc):
        c = lax.axis_index("c")
        s = lax.axis_index("s")
        w = s * _NC + c
        base = pl.multiple_of(w * _EW, 8)
        rbase = s * _NPT
        pltpu.sync_copy(zeros.at[pl.ds(rbase, _NPT)], acc.at[pl.ds(rbase, _NPT)])
        pltpu.sync_copy(idx3.at[w], idx_v)
        plsc.subcore_barrier()

        def chunk(g, carry):
            o = pl.multiple_of(g * cs, 8)
            pltpu.sync_copy(m.at[pl.ds(base + o, cs)], mb)
            pltpu.sync_copy(mb, acc.at[idx_v.at[g]], add=True)
            return carry

        lax.fori_loop(0, ncs, chunk, 0)
        plsc.subcore_barrier()
        pltpu.sync_copy(acc.at[pl.ds(rbase, _NPT)],
                        parts.at[c, pl.ds(rbase, _NPT)])

    return body


def _scatter(m, idx3, zeros, width, cs, ncs):
    f = pl.kernel(
        _make_scatter_body(width, cs, ncs),
        out_type=jax.ShapeDtypeStruct((_NC, _NP, width), _f32),
        mesh=_sc_mesh(),
        compiler_params=_SC_PARAMS,
        scratch_types=[
            pltpu.VMEM((ncs, cs), jnp.int32),
            pltpu.VMEM((cs, width), _f32),
            pltpu.VMEM_SHARED((_NP, width), _f32),
        ],
    )
    return f(m, idx3, zeros)


def _full(shape):
    return pl.BlockSpec(shape, lambda *_: (0,) * len(shape))


def _prelude_body(t_ref, teW1, teb1, teW2, teb2, filmW, filmb, ne,
                  g1p_o, beta_o, hf0_o):
    tv = t_ref[0, 0]
    temb = jax.nn.silu(tv * teW1[...] + teb1[...])
    temb = jnp.dot(temb, teW2[...], preferred_element_type=_f32) + teb2[...]
    gb = jnp.dot(temb, filmW[...], preferred_element_type=_f32) + filmb[...]
    g1p = 1.0 + gb[:, :_DIM]
    beta = gb[:, _DIM:]
    g1p_o[...] = g1p
    beta_o[...] = beta
    hf0_o[...] = ne[...] * g1p + beta


def _prelude(t11, teW1, teb1, teW2, teb2, filmW, filmb, ne):
    f = pl.pallas_call(
        _prelude_body,
        in_specs=[pl.BlockSpec(memory_space=pltpu.SMEM)] + [_full(a.shape) for a in
                  (teW1, teb1, teW2, teb2, filmW, filmb, ne)],
        out_specs=[_full((1, _DIM))] * 3,
        out_shape=(jax.ShapeDtypeStruct((1, _DIM), _f32),) * 3,
    )
    return f(t11, teW1, teb1, teW2, teb2, filmW, filmb, ne)


def _edge_mlp(feat256, dsq, W1a, W1cp, b1, W2, b2):
    dpad = jnp.concatenate([dsq, jnp.zeros((_BE, _DIM - 1), _f32)], axis=-1)
    pre = (jnp.dot(feat256, W1a, preferred_element_type=_f32)
           + jnp.dot(dpad, W1cp, preferred_element_type=_f32)
           + b1)
    mm = jnp.dot(jax.nn.silu(pre), W2, preferred_element_type=_f32) + b2
    return jax.nn.silu(mm) * (1.0 / _K)


def _edge0_body(hf0, dsq, W1a, W1cp, b1, W2, b2, m_o):
    feat = jnp.concatenate(
        [jnp.broadcast_to(hf0[...], (_BE, _DIM))] * 2, axis=-1)
    m_o[...] = _edge_mlp(feat, dsq[...], W1a[...], W1cp[...], b1[...],
                         W2[...], b2[...])


def _edge0(hf0row, dsq2, W1a, W1cp, b1, W2, b2):
    f = pl.pallas_call(
        _edge0_body,
        grid=(_E // _BE,),
        in_specs=[_full((1, _DIM)),
                  pl.BlockSpec((_BE, 1), lambda i: (i, 0)),
                  _full((2 * _DIM, _DIM)), _full((_DIM, _DIM)),
                  _full((1, _DIM)), _full((_DIM, _DIM)), _full((1, _DIM))],
        out_specs=pl.BlockSpec((_BE, _DIM), lambda i: (i, 0)),
        out_shape=jax.ShapeDtypeStruct((_E, _DIM), _f32),
    )
    return f(hf0row, dsq2, W1a, W1cp, b1, W2, b2)


def _edge_body(g1, g2, dsq, W1a, W1cp, b1, W2, b2, m_o):
    feat = jnp.concatenate([g1[...], g2[...]], axis=-1)
    m_o[...] = _edge_mlp(feat, dsq[...], W1a[...], W1cp[...], b1[...],
                         W2[...], b2[...])


def _edge(G1, G2, dsq2, W1a, W1cp, b1, W2, b2):
    f = pl.pallas_call(
        _edge_body,
        grid=(_E // _BE,),
        in_specs=[pl.BlockSpec((_BE, _DIM), lambda i: (i, 0)),
                  pl.BlockSpec((_BE, _DIM), lambda i: (i, 0)),
                  pl.BlockSpec((_BE, 1), lambda i: (i, 0)),
                  _full((2 * _DIM, _DIM)), _full((_DIM, _DIM)),
                  _full((1, _DIM)), _full((_DIM, _DIM)), _full((1, _DIM))],
        out_specs=pl.BlockSpec((_BE, _DIM), lambda i: (i, 0)),
        out_shape=jax.ShapeDtypeStruct((_E, _DIM), _f32),
    )
    return f(G1, G2, dsq2, W1a, W1cp, b1, W2, b2)


def _edgef_body(g1, g2, dsq, r0, r1, W1a, W1cp, b1, w2col, v_o):
    feat = jnp.concatenate([g1[...], g2[...]], axis=-1)
    dpad = jnp.concatenate([dsq[...], jnp.zeros((_BE, _DIM - 1), _f32)],
                           axis=-1)
    pre = (jnp.dot(feat, W1a[...], preferred_element_type=_f32)
           + jnp.dot(dpad, W1cp[...], preferred_element_type=_f32)
           + b1[...])
    w = jnp.dot(jax.nn.silu(pre), w2col[...], preferred_element_type=_f32)
    th = jnp.tanh(w / _WT)
    dij = jnp.sqrt(dsq[...] + 1e-8)
    v0 = (th * (r0[...] / dij)) * (1.0 / _K)
    v1 = (th * (r1[...] / dij)) * (1.0 / _K)
    v_o[...] = jnp.concatenate(
        [v0, v1, jnp.zeros((_BE, _DIM - 2), _f32)], axis=-1)


def _edgef(G1, G2, dsq2, r0, r1, W1a, W1cp, b1, w2col):
    f = pl.pallas_call(
        _edgef_body,
        grid=(_E // _BE,),
        in_specs=[pl.BlockSpec((_BE, _DIM), lambda i: (i, 0)),
                  pl.BlockSpec((_BE, _DIM), lambda i: (i, 0)),
                  pl.BlockSpec((_BE, 1), lambda i: (i, 0)),
                  pl.BlockSpec((_BE, 1), lambda i: (i, 0)),
                  pl.BlockSpec((_BE, 1), lambda i: (i, 0)),
                  _full((2 * _DIM, _DIM)), _full((_DIM, _DIM)),
                  _full((1, _DIM)), _full((_DIM, 1))],
        out_specs=pl.BlockSpec((_BE, _DIM), lambda i: (i, 0)),
        out_shape=jax.ShapeDtypeStruct((_E, _DIM), _f32),
    )
    return f(G1, G2, dsq2, r0, r1, W1a, W1cp, b1, w2col)


def _node_update(hf, agg, nW1, nb1, nW2, nb2, lng, lnb):
    u = jax.nn.silu(jnp.dot(jnp.concatenate([hf, agg], axis=-1), nW1,
                            preferred_element_type=_f32) + nb1)
    u = jnp.dot(u, nW2, preferred_element_type=_f32) + nb2
    z = hf + u
    mu = jnp.mean(z, axis=-1, keepdims=True)
    var = jnp.mean((z - mu) ** 2, axis=-1, keepdims=True)
    return (z - mu) / jnp.sqrt(var + 1e-5) * lng + lnb


def _node_body(hf, p0, p1, g1p, beta, nW1, nb1, nW2, nb2, lng, lnb, hf2_o):
    h = _node_update(hf[...], p0[...] + p1[...], nW1[...], nb1[...],
                     nW2[...], nb2[...], lng[...], lnb[...])
    hf2_o[...] = h * g1p[...] + beta[...]


def _node(hf, p0, p1, g1p, beta, nW1, nb1, nW2, nb2, lng, lnb):
    blk = pl.BlockSpec((_BN, _DIM), lambda i: (i, 0))
    f = pl.pallas_call(
        _node_body,
        grid=(_N // _BN,),
        in_specs=[blk, blk, blk] + [_full((1, _DIM))] * 2 +
                 [_full((2 * _DIM, _DIM)), _full((1, _DIM)),
                  _full((_DIM, _DIM)), _full((1, _DIM)),
                  _full((1, _DIM)), _full((1, _DIM))],
        out_specs=blk,
        out_shape=jax.ShapeDtypeStruct((_N, _DIM), _f32),
    )
    return f(hf, p0, p1, g1p, beta, nW1, nb1, nW2, nb2, lng, lnb)


def _nodef_body(hf, p0, p1, nW1, nb1, nW2, nb2, lng, lnb, h_o):
    h_o[...] = _node_update(hf[...], p0[...] + p1[...], nW1[...], nb1[...],
                            nW2[...], nb2[...], lng[...], lnb[...])


def _nodef(hf, p0, p1, nW1, nb1, nW2, nb2, lng, lnb):
    blk = pl.BlockSpec((_BN, _DIM), lambda i: (i, 0))
    f = pl.pallas_call(
        _nodef_body,
        grid=(_N // _BN,),
        in_specs=[blk, blk, blk] +
                 [_full((2 * _DIM, _DIM)), _full((1, _DIM)),
                  _full((_DIM, _DIM)), _full((1, _DIM)),
                  _full((1, _DIM)), _full((1, _DIM))],
        out_specs=blk,
        out_shape=jax.ShapeDtypeStruct((_N, _DIM), _f32),
    )
    return f(hf, p0, p1, nW1, nb1, nW2, nb2, lng, lnb)


def _vout_body(vp, o):
    v = vp[0] + vp[1]
    o[...] = v - jnp.sum(v, axis=0, keepdims=True) / jnp.float32(_N)


def _vout(vparts):
    f = pl.pallas_call(
        _vout_body,
        out_shape=jax.ShapeDtypeStruct((_NP, _DIM), _f32),
    )
    return f(vparts)


def kernel(x, t, edge_index, node_embedding, te_W1, te_b1, te_W2, te_b2,
           film_W, film_b, ln_g, ln_b, e_W1, e_b1, e_W2, e_b2,
           n_W1, n_b1, n_W2, n_b2, ch_W1, ch_b1, ch_W2):
    xf = x.reshape(_N, 2)
    row = edge_index[0]
    col = edge_index[1]

    rij0, rij1, dsq = _geom(xf[:, 0], xf[:, 1], row, col)
    dsq2 = dsq.reshape(_E, 1)
    r0 = rij0.reshape(_E, 1)
    r1 = rij1.reshape(_E, 1)

    idx3 = row.reshape(_NW, _NCS, _CS)
    zeros128 = jnp.zeros((_NP, _DIM), _f32)
    zpad = jnp.zeros((_DIM - 1, _DIM), _f32)

    def r2(a):
        return a.reshape(1, -1)

    g1p, beta, hf0row = _prelude(
        t.reshape(1, 1), te_W1, r2(te_b1), te_W2, r2(te_b2), film_W,
        r2(film_b), node_embedding)

    hf = jnp.broadcast_to(hf0row, (_N, _DIM))
    m = _edge0(hf0row, dsq2, e_W1[0, :2 * _DIM],
               jnp.concatenate([e_W1[0, 2 * _DIM:], zpad], 0),
               r2(e_b1[0]), e_W2[0], r2(e_b2[0]))

    h6 = None
    for l in range(_DEPTH):
        parts = _scatter(m, idx3, zeros128, _DIM, _CS, _NCS)
        if l < _DEPTH - 1:
            hf = _node(hf, parts[0], parts[1], g1p, beta,
                       n_W1[l], r2(n_b1[l]), n_W2[l], r2(n_b2[l]),
                       r2(ln_g[l]), r2(ln_b[l]))
            G1, G2 = _gather2(hf, hf, row, col)
            m = _edge(G1, G2, dsq2, e_W1[l + 1, :2 * _DIM],
                      jnp.concatenate([e_W1[l + 1, 2 * _DIM:], zpad], 0),
                      r2(e_b1[l + 1]), e_W2[l + 1], r2(e_b2[l + 1]))
        else:
            h6 = _nodef(hf, parts[0], parts[1],
                        n_W1[l], r2(n_b1[l]), n_W2[l], r2(n_b2[l]),
                        r2(ln_g[l]), r2(ln_b[l]))

    G1c, G2c = _gather2(h6, h6, row, col)
    v128 = _edgef(G1c, G2c, dsq2, r0, r1, ch_W1[:2 * _DIM],
                  jnp.concatenate([ch_W1[2 * _DIM:], zpad], 0),
                  r2(ch_b1), ch_W2)
    vparts = _scatter(v128, idx3, zeros128, _DIM, _CS, _NCS)
    o = _vout(vparts)
    return o[:_N, :2].reshape(1, _N, 2)


# speedup vs baseline: 3.2743x; 1.1021x over previous
import functools

import jax
import jax.numpy as jnp
from jax import lax
from jax.experimental import pallas as pl
from jax.experimental.pallas import tpu as pltpu
from jax.experimental.pallas import tpu_sc as plsc

_N = 10000
_E = 320000
_DIM = 128
_DEPTH = 6
_K = 32.0
_WT = 10.0

_NC = 2
_NS = 16
_NW = _NC * _NS
_EW = _E // _NW

_CG = 40
_NCG = _EW // _CG
_CS = 40
_NCS = _EW // _CS
_CSV = 80
_NCSV = _EW // _CSV
_NP = 10240
_NPT = _NP // _NS

_BE = 2560
_BN = 2000

_f32 = jnp.float32


def _sc_mesh():
    return plsc.VectorSubcoreMesh(core_axis_name="c", subcore_axis_name="s")


_SC_PARAMS = pltpu.CompilerParams(needs_layout_passes=False)


def _wid():
    return lax.axis_index("s") * _NC + lax.axis_index("c")


def _geom_body(x0, x1, row, col, rij0, rij1, dsq,
               x0_v, x1_v, row_v, col_v, o0_v, o1_v, od_v):
    base = pl.multiple_of(_wid() * _EW, 8)
    pltpu.sync_copy(x0, x0_v)
    pltpu.sync_copy(x1, x1_v)
    pltpu.sync_copy(row.at[pl.ds(base, _EW)], row_v)
    pltpu.sync_copy(col.at[pl.ds(base, _EW)], col_v)

    def step(i, carry):
        o = pl.multiple_of(i * 16, 8)
        ir = row_v[pl.ds(o, 16)]
        ic = col_v[pl.ds(o, 16)]
        r0 = plsc.load_gather(x0_v, [ir]) - plsc.load_gather(x0_v, [ic])
        r1 = plsc.load_gather(x1_v, [ir]) - plsc.load_gather(x1_v, [ic])
        o0_v[pl.ds(o, 16)] = r0
        o1_v[pl.ds(o, 16)] = r1
        od_v[pl.ds(o, 16)] = r0 * r0 + r1 * r1
        return carry

    lax.fori_loop(0, _EW // 16, step, 0)
    pltpu.sync_copy(o0_v, rij0.at[pl.ds(base, _EW)])
    pltpu.sync_copy(o1_v, rij1.at[pl.ds(base, _EW)])
    pltpu.sync_copy(od_v, dsq.at[pl.ds(base, _EW)])


def _geom(x0, x1, row, col):
    f = pl.kernel(
        _geom_body,
        out_type=(jax.ShapeDtypeStruct((_E,), _f32),) * 3,
        mesh=_sc_mesh(),
        compiler_params=_SC_PARAMS,
        scratch_types=[
            pltpu.VMEM((_N,), _f32), pltpu.VMEM((_N,), _f32),
            pltpu.VMEM((_EW,), jnp.int32), pltpu.VMEM((_EW,), jnp.int32),
            pltpu.VMEM((_EW,), _f32), pltpu.VMEM((_EW,), _f32),
            pltpu.VMEM((_EW,), _f32),
        ],
    )
    return f(x0, x1, row, col)


def _gather_body(t1, t2, row, col, g1, g2, row_v, col_v,
                 bA1, bA2, bB1, bB2, sgA, sgB, soA, soB):
    base = pl.multiple_of(_wid() * _EW, 8)
    pltpu.sync_copy(row.at[pl.ds(base, _EW)], row_v)
    pltpu.sync_copy(col.at[pl.ds(base, _EW)], col_v)

    def start_g(g, b1, b2, sg):
        o = pl.multiple_of((g % _NCG) * _CG, 8)
        pltpu.async_copy(t1.at[row_v.at[pl.ds(o, _CG)]], b1, sg)
        pltpu.async_copy(t2.at[col_v.at[pl.ds(o, _CG)]], b2, sg)

    def wait_g(g, b1, b2, sg):
        o = pl.multiple_of((g % _NCG) * _CG, 8)
        pltpu.make_async_copy(t1.at[row_v.at[pl.ds(o, _CG)]], b1, sg).wait()
        pltpu.make_async_copy(t2.at[col_v.at[pl.ds(o, _CG)]], b2, sg).wait()

    def start_o(g, b1, b2, so):
        o = pl.multiple_of(g * _CG, 8)
        pltpu.async_copy(b1, g1.at[pl.ds(base + o, _CG)], so)
        pltpu.async_copy(b2, g2.at[pl.ds(base + o, _CG)], so)

    def wait_o(g, b1, b2, so):
        o = pl.multiple_of(g * _CG, 8)
        pltpu.make_async_copy(b1, g1.at[pl.ds(base + o, _CG)], so).wait()
        pltpu.make_async_copy(b2, g2.at[pl.ds(base + o, _CG)], so).wait()

    start_g(0, bA1, bA2, sgA)
    start_g(1, bB1, bB2, sgB)

    def pair(p, carry):
        cA = 2 * p
        cB = 2 * p + 1
        wait_g(cA, bA1, bA2, sgA)
        start_o(cA, bA1, bA2, soA)
        wait_g(cB, bB1, bB2, sgB)
        start_o(cB, bB1, bB2, soB)
        wait_o(cA, bA1, bA2, soA)
        start_g(cA + 2, bA1, bA2, sgA)
        wait_o(cB, bB1, bB2, soB)
        start_g(cB + 2, bB1, bB2, sgB)
        return carry

    lax.fori_loop(0, _NCG // 2, pair, 0)
    wait_g(0, bA1, bA2, sgA)
    wait_g(1, bB1, bB2, sgB)


def _gather2(t1, t2, row, col):
    f = pl.kernel(
        _gather_body,
        out_type=(jax.ShapeDtypeStruct((_E, _DIM), _f32),) * 2,
        mesh=_sc_mesh(),
        compiler_params=_SC_PARAMS,
        scratch_types=[
            pltpu.VMEM((_EW,), jnp.int32), pltpu.VMEM((_EW,), jnp.int32),
            pltpu.VMEM((_CG, _DIM), _f32), pltpu.VMEM((_CG, _DIM), _f32),
            pltpu.VMEM((_CG, _DIM), _f32), pltpu.VMEM((_CG, _DIM), _f32),
            pltpu.SemaphoreType.DMA, pltpu.SemaphoreType.DMA,
            pltpu.SemaphoreType.DMA, pltpu.SemaphoreType.DMA,
        ],
    )
    return f(t1, t2, row, col)


def _make_scatter_body(width, cs, ncs):
    def body(m, idx3, zeros, parts, idx_v, mb, mb2, acc, sem):
        c = lax.axis_index("c")
        s = lax.axis_index("s")
        w = s * _NC + c
        base = pl.multiple_of(w * _EW, 8)
        rbase = s * _NPT
        pltpu.sync_copy(zeros.at[pl.ds(rbase, _NPT)], acc.at[pl.ds(rbase, _NPT)])
        pltpu.sync_copy(idx3.at[w], idx_v)
        plsc.subcore_barrier()

        def start_in(g, buf):
            o = pl.multiple_of((g % ncs) * cs, 8)
            pltpu.async_copy(m.at[pl.ds(base + o, cs)], buf, sem)

        def wait_in(g, buf):
            o = pl.multiple_of((g % ncs) * cs, 8)
            pltpu.make_async_copy(m.at[pl.ds(base + o, cs)], buf, sem).wait()

        start_in(0, mb)

        def chunk2(p, carry):
            cA = 2 * p
            cB = 2 * p + 1
            wait_in(cA, mb)
            start_in(cB, mb2)
            pltpu.sync_copy(mb, acc.at[idx_v.at[cA]], add=True)
            wait_in(cB, mb2)
            start_in(cA + 2, mb)
            pltpu.sync_copy(mb2, acc.at[idx_v.at[cB]], add=True)
            return carry

        lax.fori_loop(0, ncs // 2, chunk2, 0)
        wait_in(0, mb)
        plsc.subcore_barrier()
        pltpu.sync_copy(acc.at[pl.ds(rbase, _NPT)],
                        parts.at[c, pl.ds(rbase, _NPT)])

    return body


def _scatter(m, idx3, zeros, width, cs, ncs):
    f = pl.kernel(
        _make_scatter_body(width, cs, ncs),
        out_type=jax.ShapeDtypeStruct((_NC, _NP, width), _f32),
        mesh=_sc_mesh(),
        compiler_params=_SC_PARAMS,
        scratch_types=[
            pltpu.VMEM((ncs, cs), jnp.int32),
            pltpu.VMEM((cs, width), _f32),
            pltpu.VMEM((cs, width), _f32),
            pltpu.VMEM_SHARED((_NP, width), _f32),
            pltpu.SemaphoreType.DMA,
        ],
    )
    return f(m, idx3, zeros)


def _full(shape):
    return pl.BlockSpec(shape, lambda *_: (0,) * len(shape))


def _prelude_body(t_ref, teW1, teb1, teW2, teb2, filmW, filmb, ne,
                  g1p_o, beta_o, hf0_o):
    tv = t_ref[0, 0]
    temb = jax.nn.silu(tv * teW1[...] + teb1[...])
    temb = jnp.dot(temb, teW2[...], preferred_element_type=_f32) + teb2[...]
    gb = jnp.dot(temb, filmW[...], preferred_element_type=_f32) + filmb[...]
    g1p = 1.0 + gb[:, :_DIM]
    beta = gb[:, _DIM:]
    g1p_o[...] = g1p
    beta_o[...] = beta
    hf0_o[...] = ne[...] * g1p + beta


def _prelude(t11, teW1, teb1, teW2, teb2, filmW, filmb, ne):
    f = pl.pallas_call(
        _prelude_body,
        in_specs=[pl.BlockSpec(memory_space=pltpu.SMEM)] + [_full(a.shape) for a in
                  (teW1, teb1, teW2, teb2, filmW, filmb, ne)],
        out_specs=[_full((1, _DIM))] * 3,
        out_shape=(jax.ShapeDtypeStruct((1, _DIM), _f32),) * 3,
    )
    return f(t11, teW1, teb1, teW2, teb2, filmW, filmb, ne)


def _edge_mlp(feat256, dsq, W1a, W1cp, b1, W2, b2):
    dpad = jnp.concatenate([dsq, jnp.zeros((_BE, _DIM - 1), _f32)], axis=-1)
    pre = (jnp.dot(feat256, W1a, preferred_element_type=_f32)
           + jnp.dot(dpad, W1cp, preferred_element_type=_f32)
           + b1)
    mm = jnp.dot(jax.nn.silu(pre), W2, preferred_element_type=_f32) + b2
    return jax.nn.silu(mm) * (1.0 / _K)


def _edge0_body(hf0, dsq, W1a, W1cp, b1, W2, b2, m_o):
    feat = jnp.concatenate(
        [jnp.broadcast_to(hf0[...], (_BE, _DIM))] * 2, axis=-1)
    m_o[...] = _edge_mlp(feat, dsq[...], W1a[...], W1cp[...], b1[...],
                         W2[...], b2[...])


def _edge0(hf0row, dsq2, W1a, W1cp, b1, W2, b2):
    f = pl.pallas_call(
        _edge0_body,
        grid=(_E // _BE,),
        in_specs=[_full((1, _DIM)),
                  pl.BlockSpec((_BE, 1), lambda i: (i, 0)),
                  _full((2 * _DIM, _DIM)), _full((_DIM, _DIM)),
                  _full((1, _DIM)), _full((_DIM, _DIM)), _full((1, _DIM))],
        out_specs=pl.BlockSpec((_BE, _DIM), lambda i: (i, 0)),
        out_shape=jax.ShapeDtypeStruct((_E, _DIM), _f32),
    )
    return f(hf0row, dsq2, W1a, W1cp, b1, W2, b2)


def _edge_body(g1, g2, dsq, W1a, W1cp, b1, W2, b2, m_o):
    feat = jnp.concatenate([g1[...], g2[...]], axis=-1)
    m_o[...] = _edge_mlp(feat, dsq[...], W1a[...], W1cp[...], b1[...],
                         W2[...], b2[...])


def _edge(G1, G2, dsq2, W1a, W1cp, b1, W2, b2):
    f = pl.pallas_call(
        _edge_body,
        grid=(_E // _BE,),
        in_specs=[pl.BlockSpec((_BE, _DIM), lambda i: (i, 0)),
                  pl.BlockSpec((_BE, _DIM), lambda i: (i, 0)),
                  pl.BlockSpec((_BE, 1), lambda i: (i, 0)),
                  _full((2 * _DIM, _DIM)), _full((_DIM, _DIM)),
                  _full((1, _DIM)), _full((_DIM, _DIM)), _full((1, _DIM))],
        out_specs=pl.BlockSpec((_BE, _DIM), lambda i: (i, 0)),
        out_shape=jax.ShapeDtypeStruct((_E, _DIM), _f32),
    )
    return f(G1, G2, dsq2, W1a, W1cp, b1, W2, b2)


def _edgef_body(g1, g2, dsq, r0, r1, W1a, W1cp, b1, w2col, v_o):
    feat = jnp.concatenate([g1[...], g2[...]], axis=-1)
    dpad = jnp.concatenate([dsq[...], jnp.zeros((_BE, _DIM - 1), _f32)],
                           axis=-1)
    pre = (jnp.dot(feat, W1a[...], preferred_element_type=_f32)
           + jnp.dot(dpad, W1cp[...], preferred_element_type=_f32)
           + b1[...])
    w = jnp.dot(jax.nn.silu(pre), w2col[...], preferred_element_type=_f32)
    th = jnp.tanh(w / _WT)
    dij = jnp.sqrt(dsq[...] + 1e-8)
    v0 = (th * (r0[...] / dij)) * (1.0 / _K)
    v1 = (th * (r1[...] / dij)) * (1.0 / _K)
    v_o[...] = jnp.concatenate(
        [v0, v1, jnp.zeros((_BE, _DIM - 2), _f32)], axis=-1)


def _edgef(G1, G2, dsq2, r0, r1, W1a, W1cp, b1, w2col):
    f = pl.pallas_call(
        _edgef_body,
        grid=(_E // _BE,),
        in_specs=[pl.BlockSpec((_BE, _DIM), lambda i: (i, 0)),
                  pl.BlockSpec((_BE, _DIM), lambda i: (i, 0)),
                  pl.BlockSpec((_BE, 1), lambda i: (i, 0)),
                  pl.BlockSpec((_BE, 1), lambda i: (i, 0)),
                  pl.BlockSpec((_BE, 1), lambda i: (i, 0)),
                  _full((2 * _DIM, _DIM)), _full((_DIM, _DIM)),
                  _full((1, _DIM)), _full((_DIM, 1))],
        out_specs=pl.BlockSpec((_BE, _DIM), lambda i: (i, 0)),
        out_shape=jax.ShapeDtypeStruct((_E, _DIM), _f32),
    )
    return f(G1, G2, dsq2, r0, r1, W1a, W1cp, b1, w2col)


def _node_update(hf, agg, nW1, nb1, nW2, nb2, lng, lnb):
    u = jax.nn.silu(jnp.dot(jnp.concatenate([hf, agg], axis=-1), nW1,
                            preferred_element_type=_f32) + nb1)
    u = jnp.dot(u, nW2, preferred_element_type=_f32) + nb2
    z = hf + u
    mu = jnp.mean(z, axis=-1, keepdims=True)
    var = jnp.mean((z - mu) ** 2, axis=-1, keepdims=True)
    return (z - mu) / jnp.sqrt(var + 1e-5) * lng + lnb


def _node_body(hf, p0, p1, g1p, beta, nW1, nb1, nW2, nb2, lng, lnb, hf2_o):
    h = _node_update(hf[...], p0[...] + p1[...], nW1[...], nb1[...],
                     nW2[...], nb2[...], lng[...], lnb[...])
    hf2_o[...] = h * g1p[...] + beta[...]


def _node(hf, p0, p1, g1p, beta, nW1, nb1, nW2, nb2, lng, lnb):
    blk = pl.BlockSpec((_BN, _DIM), lambda i: (i, 0))
    f = pl.pallas_call(
        _node_body,
        grid=(_N // _BN,),
        in_specs=[blk, blk, blk] + [_full((1, _DIM))] * 2 +
                 [_full((2 * _DIM, _DIM)), _full((1, _DIM)),
                  _full((_DIM, _DIM)), _full((1, _DIM)),
                  _full((1, _DIM)), _full((1, _DIM))],
        out_specs=blk,
        out_shape=jax.ShapeDtypeStruct((_N, _DIM), _f32),
    )
    return f(hf, p0, p1, g1p, beta, nW1, nb1, nW2, nb2, lng, lnb)


def _nodef_body(hf, p0, p1, nW1, nb1, nW2, nb2, lng, lnb, h_o):
    h_o[...] = _node_update(hf[...], p0[...] + p1[...], nW1[...], nb1[...],
                            nW2[...], nb2[...], lng[...], lnb[...])


def _nodef(hf, p0, p1, nW1, nb1, nW2, nb2, lng, lnb):
    blk = pl.BlockSpec((_BN, _DIM), lambda i: (i, 0))
    f = pl.pallas_call(
        _nodef_body,
        grid=(_N // _BN,),
        in_specs=[blk, blk, blk] +
                 [_full((2 * _DIM, _DIM)), _full((1, _DIM)),
                  _full((_DIM, _DIM)), _full((1, _DIM)),
                  _full((1, _DIM)), _full((1, _DIM))],
        out_specs=blk,
        out_shape=jax.ShapeDtypeStruct((_N, _DIM), _f32),
    )
    return f(hf, p0, p1, nW1, nb1, nW2, nb2, lng, lnb)


def _vout_body(vp, o):
    v = vp[0] + vp[1]
    o[...] = v - jnp.sum(v, axis=0, keepdims=True) / jnp.float32(_N)


def _vout(vparts):
    f = pl.pallas_call(
        _vout_body,
        out_shape=jax.ShapeDtypeStruct((_NP, _DIM), _f32),
    )
    return f(vparts)


def kernel(x, t, edge_index, node_embedding, te_W1, te_b1, te_W2, te_b2,
           film_W, film_b, ln_g, ln_b, e_W1, e_b1, e_W2, e_b2,
           n_W1, n_b1, n_W2, n_b2, ch_W1, ch_b1, ch_W2):
    xf = x.reshape(_N, 2)
    row = edge_index[0]
    col = edge_index[1]

    rij0, rij1, dsq = _geom(xf[:, 0], xf[:, 1], row, col)
    dsq2 = dsq.reshape(_E, 1)
    r0 = rij0.reshape(_E, 1)
    r1 = rij1.reshape(_E, 1)

    idx3 = row.reshape(_NW, _NCS, _CS)
    zeros128 = jnp.zeros((_NP, _DIM), _f32)
    zpad = jnp.zeros((_DIM - 1, _DIM), _f32)

    def r2(a):
        return a.reshape(1, -1)

    g1p, beta, hf0row = _prelude(
        t.reshape(1, 1), te_W1, r2(te_b1), te_W2, r2(te_b2), film_W,
        r2(film_b), node_embedding)

    hf = jnp.broadcast_to(hf0row, (_N, _DIM))
    m = _edge0(hf0row, dsq2, e_W1[0, :2 * _DIM],
               jnp.concatenate([e_W1[0, 2 * _DIM:], zpad], 0),
               r2(e_b1[0]), e_W2[0], r2(e_b2[0]))

    h6 = None
    for l in range(_DEPTH):
        parts = _scatter(m, idx3, zeros128, _DIM, _CS, _NCS)
        if l < _DEPTH - 1:
            hf = _node(hf, parts[0], parts[1], g1p, beta,
                       n_W1[l], r2(n_b1[l]), n_W2[l], r2(n_b2[l]),
                       r2(ln_g[l]), r2(ln_b[l]))
            G1, G2 = _gather2(hf, hf, row, col)
            m = _edge(G1, G2, dsq2, e_W1[l + 1, :2 * _DIM],
                      jnp.concatenate([e_W1[l + 1, 2 * _DIM:], zpad], 0),
                      r2(e_b1[l + 1]), e_W2[l + 1], r2(e_b2[l + 1]))
        else:
            h6 = _nodef(hf, parts[0], parts[1],
                        n_W1[l], r2(n_b1[l]), n_W2[l], r2(n_b2[l]),
                        r2(ln_g[l]), r2(ln_b[l]))

    G1c, G2c = _gather2(h6, h6, row, col)
    v128 = _edgef(G1c, G2c, dsq2, r0, r1, ch_W1[:2 * _DIM],
                  jnp.concatenate([ch_W1[2 * _DIM:], zpad], 0),
                  r2(ch_b1), ch_W2)
    vparts = _scatter(v128, idx3, zeros128, _DIM, _CS, _NCS)
    o = _vout(vparts)
    return o[:_N, :2].reshape(1, _N, 2)
